# Initial kernel scaffold; baseline (speedup 1.0000x reference)
import jax
import jax.numpy as jnp
from jax.experimental import pallas as pl


def kernel(x, pos, edge_index, Wq, bq, Wk, bk, Wv, bv, W1, b1, gamma, beta):
    raise NotImplementedError("write your pallas kernel here")


# speedup vs baseline: 1.5954x; 1.5954x over previous
import functools

import jax
import jax.numpy as jnp
from jax import lax
from jax.experimental import pallas as pl
from jax.experimental.pallas import tpu as pltpu
from jax.experimental.pallas import tpu_sc as plsc

NC = 2
NS = 16
NW = NC * NS
CH = 128
BLK = 512


def _node_values(x, Wv, bv, W1v, b1):
    n, c = x.shape
    h = W1v.shape[1]
    nb = pl.cdiv(n, BLK)

    def body(x_ref, wv_ref, bv_ref, w1v_ref, b1_ref, v_ref, weff_ref, beff_ref):
        @pl.when(pl.program_id(0) == 0)
        def _():
            weff_ref[...] = jnp.dot(wv_ref[...], w1v_ref[...],
                                    preferred_element_type=jnp.float32)
            beff_ref[...] = jnp.dot(bv_ref[...], w1v_ref[...],
                                    preferred_element_type=jnp.float32) + b1_ref[...]

        v_ref[...] = jnp.dot(x_ref[...], weff_ref[...],
                             preferred_element_type=jnp.float32) + beff_ref[...]

    return pl.pallas_call(
        body,
        grid=(nb,),
        in_specs=[
            pl.BlockSpec((BLK, c), lambda i: (i, 0)),
            pl.BlockSpec((c, c), lambda i: (0, 0)),
            pl.BlockSpec((1, c), lambda i: (0, 0)),
            pl.BlockSpec((c, h), lambda i: (0, 0)),
            pl.BlockSpec((1, h), lambda i: (0, 0)),
        ],
        out_specs=pl.BlockSpec((BLK, h), lambda i: (i, 0)),
        out_shape=jax.ShapeDtypeStruct((n, h), jnp.float32),
        scratch_shapes=[pltpu.VMEM((c, h), jnp.float32),
                        pltpu.VMEM((1, h), jnp.float32)],
    )(x, Wv, bv, W1v, b1)


def _sc_mesh():
    return plsc.VectorSubcoreMesh(core_axis_name="c", subcore_axis_name="s")


def _worker_base(epw):
    wid = lax.axis_index("s") * NC + lax.axis_index("c")
    return wid * epw


def _edge_geometry(pos_cols, dst_p, ep, e_real):
    epw = ep // NW
    nch = epw // CH

    @functools.partial(
        pl.kernel,
        mesh=_sc_mesh(),
        compiler_params=pltpu.CompilerParams(needs_layout_passes=False),
        out_type=[jax.ShapeDtypeStruct((ep,), jnp.float32),
                  jax.ShapeDtypeStruct((4, ep), jnp.float32)],
        scratch_types=[pltpu.VMEM((epw,), jnp.int32)]
        + [pltpu.VMEM((epw,), jnp.float32) for _ in range(7)]
        + [pltpu.VMEM((epw,), jnp.float32),
           pltpu.VMEM((4, epw), jnp.float32),
           pltpu.SemaphoreType.DMA],
    )
    def k(px_hbm, py_hbm, pz_hbm, pw_hbm, dst_hbm, pe_hbm, rel_hbm,
          idx_v, pjx_v, pjy_v, pjz_v, pjw_v, pix_v, piy_v, piz_v,
          pe_v, rel_v, sem):
        base = _worker_base(epw)
        pltpu.sync_copy(dst_hbm.at[pl.ds(base, epw)], idx_v)
        pltpu.sync_copy(px_hbm.at[pl.ds(base, epw)], pix_v)
        pltpu.sync_copy(py_hbm.at[pl.ds(base, epw)], piy_v)
        pltpu.sync_copy(pz_hbm.at[pl.ds(base, epw)], piz_v)
        cps = []
        for kk in range(nch):
            isl = idx_v.at[pl.ds(kk * CH, CH)]
            osl = pl.ds(kk * CH, CH)
            for tab, dv in ((px_hbm, pjx_v), (py_hbm, pjy_v),
                            (pz_hbm, pjz_v), (pw_hbm, pjw_v)):
                cps.append(pltpu.async_copy(tab.at[isl], dv.at[osl], sem))
        for cp in cps:
            cp.wait()

        def chunk(j, carry):
            r = j * 16
            sl = pl.ds(r, 16)
            rows = lax.iota(jnp.int32, 16) + r
            rx = pjx_v[sl] - pix_v[sl]
            ry = pjy_v[sl] - piy_v[sl]
            rz = pjz_v[sl] - piz_v[sl]
            pe = jnp.maximum(jnp.maximum(jnp.abs(rx), jnp.abs(ry)), jnp.abs(rz))
            pe = jnp.where(rows + base < e_real, pe, 0.0)
            pe_v[sl] = pe
            rel_v[0, sl] = rx
            rel_v[1, sl] = ry
            rel_v[2, sl] = rz
            rel_v[3, sl] = pjw_v[sl]
            return carry

        lax.fori_loop(0, epw // 16, chunk, 0)
        pltpu.sync_copy(pe_v, pe_hbm.at[pl.ds(base, epw)])
        pltpu.sync_copy(rel_v, rel_hbm.at[:, pl.ds(base, epw)])

    return k(*pos_cols, dst_p)


def _segment_max(src_p, pe, ep):
    npw = ep // NW
    half = ep // 2

    @functools.partial(
        pl.kernel,
        mesh=_sc_mesh(),
        compiler_params=pltpu.CompilerParams(needs_layout_passes=False),
        out_type=jax.ShapeDtypeStruct((ep,), jnp.float32),
        scratch_types=[pltpu.VMEM((half,), jnp.int32),
                       pltpu.VMEM((half,), jnp.float32),
                       pltpu.VMEM((16 * npw,), jnp.float32),
                       pltpu.VMEM((npw,), jnp.float32)],
    )
    def k(src_hbm, pe_hbm, seg_hbm, src_v, pe_v, seg16_v, seg_v):
        n0 = _worker_base(npw)
        iota = lax.iota(jnp.int32, 16)
        plane = iota * npw

        def zi(j, carry):
            seg16_v[pl.ds(j * 16, 16)] = jnp.zeros((16,), jnp.float32)
            return carry

        lax.fori_loop(0, npw, zi, 0)

        for hb in range(2):
            pltpu.sync_copy(src_hbm.at[pl.ds(hb * half, half)], src_v)
            pltpu.sync_copy(pe_hbm.at[pl.ds(hb * half, half)], pe_v)

            def chunk(j, carry):
                sl = pl.ds(j * 16, 16)
                loc = src_v[sl] - n0
                v = pe_v[sl]
                m = (loc >= 0) & (loc < npw)
                locc = jnp.clip(loc, 0, npw - 1) + plane
                old = plsc.load_gather(seg16_v, [locc])
                plsc.store_scatter(seg16_v, [locc], jnp.maximum(old, v), mask=m)
                return carry

            lax.fori_loop(0, half // 16, chunk, 0)

        def red(j, carry):
            acc = seg16_v[pl.ds(j * 16, 16)]
            for p in range(1, 16):
                acc = jnp.maximum(acc, seg16_v[pl.ds(p * npw + j * 16, 16)])
            seg_v[pl.ds(j * 16, 16)] = acc
            return carry

        lax.fori_loop(0, npw // 16, red, 0)
        pltpu.sync_copy(seg_v, seg_hbm.at[pl.ds(n0, npw)])

    return k(src_p, pe)


def _edge_assemble(seg, src_p, dst_p, rel, v_nodes, ep):
    epw = ep // NW
    nch = epw // CH
    h = v_nodes.shape[1]

    @functools.partial(
        pl.kernel,
        mesh=_sc_mesh(),
        compiler_params=pltpu.CompilerParams(needs_layout_passes=False),
        out_type=[jax.ShapeDtypeStruct((4, ep), jnp.float32),
                  jax.ShapeDtypeStruct((ep, h), jnp.float32)],
        scratch_types=[pltpu.VMEM((epw,), jnp.int32),
                       pltpu.VMEM((epw,), jnp.int32),
                       pltpu.VMEM((epw,), jnp.float32),
                       pltpu.VMEM((4, epw), jnp.float32),
                       pltpu.VMEM((4, epw), jnp.float32),
                       pltpu.VMEM((CH, 128), jnp.float32),
                       pltpu.VMEM((CH, 128), jnp.float32),
                       pltpu.SemaphoreType.DMA],
    )
    def k(seg_hbm, src_hbm, dst_hbm, rel_hbm, v_hbm, geom_hbm, vg_hbm,
          sidx_v, didx_v, maxd_v, rel_v, geom_v, buf0, buf1, sem):
        base = _worker_base(epw)
        pltpu.sync_copy(src_hbm.at[pl.ds(base, epw)], sidx_v)
        pltpu.sync_copy(dst_hbm.at[pl.ds(base, epw)], didx_v)
        pltpu.sync_copy(rel_hbm.at[:, pl.ds(base, epw)], rel_v)
        cps = [pltpu.async_copy(seg_hbm.at[sidx_v.at[pl.ds(kk * CH, CH)]],
                                maxd_v.at[pl.ds(kk * CH, CH)], sem)
               for kk in range(nch)]
        for cp in cps:
            cp.wait()

        def chunk(j, carry):
            sl = pl.ds(j * 16, 16)
            inv = 1.0 / jnp.maximum(maxd_v[sl], 1e-8)
            geom_v[0, sl] = rel_v[0, sl] * inv
            geom_v[1, sl] = rel_v[1, sl] * inv
            geom_v[2, sl] = rel_v[2, sl] * inv
            geom_v[3, sl] = rel_v[3, sl]
            return carry

        lax.fori_loop(0, epw // 16, chunk, 0)
        pltpu.sync_copy(geom_v, geom_hbm.at[:, pl.ds(base, epw)])

        bufs = (buf0, buf1)
        handles = {}
        for kk in range(nch):
            b = bufs[kk % 2]
            if kk >= 2:
                handles[kk - 2].wait()
                pltpu.sync_copy(b, vg_hbm.at[pl.ds(base + (kk - 2) * CH, CH)])
            handles[kk] = pltpu.async_copy(
                v_hbm.at[didx_v.at[pl.ds(kk * CH, CH)]], b, sem)
        for kk in (nch - 2, nch - 1):
            handles[kk].wait()
            pltpu.sync_copy(bufs[kk % 2], vg_hbm.at[pl.ds(base + kk * CH, CH)])

    return k(seg, src_p, dst_p, rel, v_nodes)


def _stats(vg, geom, W1g, e_real):
    ep, h = vg.shape
    nb = ep // BLK

    def body(vg_ref, g_ref, w1g_ref, h_ref, s_ref, q_ref):
        i = pl.program_id(0)
        acc = vg_ref[...]
        acc += lax.dot_general(g_ref[...], w1g_ref[...],
                               (((0,), (0,)), ((), ())),
                               preferred_element_type=jnp.float32)
        acc = jnp.maximum(acc, 0.0)
        rows = i * BLK + lax.broadcasted_iota(jnp.int32, (BLK, 1), 0)
        acc = jnp.where(rows < e_real, acc, 0.0)
        h_ref[...] = acc

        @pl.when(i == 0)
        def _():
            s_ref[...] = jnp.zeros_like(s_ref)
            q_ref[...] = jnp.zeros_like(q_ref)

        s_ref[...] += jnp.sum(acc, axis=0, keepdims=True)
        q_ref[...] += jnp.sum(acc * acc, axis=0, keepdims=True)

    return pl.pallas_call(
        body,
        grid=(nb,),
        in_specs=[
            pl.BlockSpec((BLK, h), lambda i: (i, 0)),
            pl.BlockSpec((4, BLK), lambda i: (0, i)),
            pl.BlockSpec((4, h), lambda i: (0, 0)),
        ],
        out_specs=[
            pl.BlockSpec((BLK, h), lambda i: (i, 0)),
            pl.BlockSpec((1, h), lambda i: (0, 0)),
            pl.BlockSpec((1, h), lambda i: (0, 0)),
        ],
        out_shape=[jax.ShapeDtypeStruct((ep, h), jnp.float32),
                   jax.ShapeDtypeStruct((1, h), jnp.float32),
                   jax.ShapeDtypeStruct((1, h), jnp.float32)],
    )(vg, geom, W1g)


def _normalize(hmat, s, q, gamma, beta, e_real):
    ep, h = hmat.shape
    nb = ep // BLK
    inv_n = 1.0 / e_real

    def body(h_ref, s_ref, q_ref, gam_ref, bet_ref, o_ref):
        mean = s_ref[...] * inv_n
        var = q_ref[...] * inv_n - mean * mean
        scale = gam_ref[...] / jnp.sqrt(var + 1e-5)
        o_ref[...] = (h_ref[...] - mean) * scale + bet_ref[...]

    return pl.pallas_call(
        body,
        grid=(nb,),
        in_specs=[
            pl.BlockSpec((BLK, h), lambda i: (i, 0)),
            pl.BlockSpec((1, h), lambda i: (0, 0)),
            pl.BlockSpec((1, h), lambda i: (0, 0)),
            pl.BlockSpec((1, h), lambda i: (0, 0)),
            pl.BlockSpec((1, h), lambda i: (0, 0)),
        ],
        out_specs=pl.BlockSpec((BLK, h), lambda i: (i, 0)),
        out_shape=jax.ShapeDtypeStruct((ep, h), jnp.float32),
    )(hmat, s, q, gamma, beta)


def kernel(x, pos, edge_index, Wq, bq, Wk, bk, Wv, bv, W1, b1, gamma, beta):
    n, c = x.shape
    e = edge_index.shape[1]
    h = W1.shape[1]

    src = edge_index[0].astype(jnp.int32)
    dst = edge_index[1].astype(jnp.int32)

    epw = -(-e // (NW * CH)) * CH
    ep = NW * epw
    src_p = jnp.concatenate([src, jnp.zeros((ep - e,), jnp.int32)])
    dst_p = jnp.concatenate([dst, jnp.zeros((ep - e,), jnp.int32)])
    zpad = jnp.zeros((ep - n,), jnp.float32)
    pos_cols = [jnp.concatenate([pos[:, i], zpad]) for i in range(4)]

    W1v = W1[:c]
    W1g = W1[c:]

    v_nodes = _node_values(x, Wv, bv[None, :], W1v, b1[None, :])
    pe, rel = _edge_geometry(pos_cols, dst_p, ep, e)
    seg = _segment_max(src_p, pe, ep)
    geom, vg = _edge_assemble(seg, src_p, dst_p, rel, v_nodes, ep)
    hmat, s, q = _stats(vg, geom, W1g, e)
    out = _normalize(hmat, s, q, gamma[None, :], beta[None, :], e)
    return out[:e]


# speedup vs baseline: 1.7223x; 1.0795x over previous
import functools

import jax
import jax.numpy as jnp
from jax import lax
from jax.experimental import pallas as pl
from jax.experimental.pallas import tpu as pltpu
from jax.experimental.pallas import tpu_sc as plsc

NC = 2
NS = 16
NW = NC * NS
CH = 128
BLK = 512


def _node_values(x, Wv, bv, W1v, b1):
    n, c = x.shape
    h = W1v.shape[1]
    nb = pl.cdiv(n, BLK)

    def body(x_ref, wv_ref, bv_ref, w1v_ref, b1_ref, v_ref, weff_ref, beff_ref):
        @pl.when(pl.program_id(0) == 0)
        def _():
            weff_ref[...] = jnp.dot(wv_ref[...], w1v_ref[...],
                                    preferred_element_type=jnp.float32)
            beff_ref[...] = jnp.dot(bv_ref[...], w1v_ref[...],
                                    preferred_element_type=jnp.float32) + b1_ref[...]

        v_ref[...] = jnp.dot(x_ref[...], weff_ref[...],
                             preferred_element_type=jnp.float32) + beff_ref[...]

    return pl.pallas_call(
        body,
        grid=(nb,),
        in_specs=[
            pl.BlockSpec((BLK, c), lambda i: (i, 0)),
            pl.BlockSpec((c, c), lambda i: (0, 0)),
            pl.BlockSpec((1, c), lambda i: (0, 0)),
            pl.BlockSpec((c, h), lambda i: (0, 0)),
            pl.BlockSpec((1, h), lambda i: (0, 0)),
        ],
        out_specs=pl.BlockSpec((BLK, h), lambda i: (i, 0)),
        out_shape=jax.ShapeDtypeStruct((n, h), jnp.float32),
        scratch_shapes=[pltpu.VMEM((c, h), jnp.float32),
                        pltpu.VMEM((1, h), jnp.float32)],
    )(x, Wv, bv, W1v, b1)


def _sc_mesh():
    return plsc.VectorSubcoreMesh(core_axis_name="c", subcore_axis_name="s")


def _worker_base(epw):
    wid = lax.axis_index("s") * NC + lax.axis_index("c")
    return wid * epw


def _edge_geometry(pos_cols, dst_p, ep, e_real):
    epw = ep // NW
    nch = epw // CH

    @functools.partial(
        pl.kernel,
        mesh=_sc_mesh(),
        compiler_params=pltpu.CompilerParams(needs_layout_passes=False),
        out_type=[jax.ShapeDtypeStruct((ep,), jnp.float32),
                  jax.ShapeDtypeStruct((4, ep), jnp.float32)],
        scratch_types=[pltpu.VMEM((epw,), jnp.int32)]
        + [pltpu.VMEM((epw,), jnp.float32) for _ in range(7)]
        + [pltpu.VMEM((epw,), jnp.float32),
           pltpu.VMEM((4, epw), jnp.float32),
           pltpu.SemaphoreType.DMA],
    )
    def k(px_hbm, py_hbm, pz_hbm, pw_hbm, dst_hbm, pe_hbm, rel_hbm,
          idx_v, pjx_v, pjy_v, pjz_v, pjw_v, pix_v, piy_v, piz_v,
          pe_v, rel_v, sem):
        base = _worker_base(epw)
        pltpu.sync_copy(dst_hbm.at[pl.ds(base, epw)], idx_v)
        pltpu.sync_copy(px_hbm.at[pl.ds(base, epw)], pix_v)
        pltpu.sync_copy(py_hbm.at[pl.ds(base, epw)], piy_v)
        pltpu.sync_copy(pz_hbm.at[pl.ds(base, epw)], piz_v)
        cps = []
        for kk in range(nch):
            isl = idx_v.at[pl.ds(kk * CH, CH)]
            osl = pl.ds(kk * CH, CH)
            for tab, dv in ((px_hbm, pjx_v), (py_hbm, pjy_v),
                            (pz_hbm, pjz_v), (pw_hbm, pjw_v)):
                cps.append(pltpu.async_copy(tab.at[isl], dv.at[osl], sem))
        for cp in cps:
            cp.wait()

        def chunk(j, carry):
            r = j * 16
            sl = pl.ds(r, 16)
            rows = lax.iota(jnp.int32, 16) + r
            rx = pjx_v[sl] - pix_v[sl]
            ry = pjy_v[sl] - piy_v[sl]
            rz = pjz_v[sl] - piz_v[sl]
            pe = jnp.maximum(jnp.maximum(jnp.abs(rx), jnp.abs(ry)), jnp.abs(rz))
            pe = jnp.where(rows + base < e_real, pe, 0.0)
            pe_v[sl] = pe
            rel_v[0, sl] = rx
            rel_v[1, sl] = ry
            rel_v[2, sl] = rz
            rel_v[3, sl] = pjw_v[sl]
            return carry

        lax.fori_loop(0, epw // 16, chunk, 0)
        pltpu.sync_copy(pe_v, pe_hbm.at[pl.ds(base, epw)])
        pltpu.sync_copy(rel_v, rel_hbm.at[:, pl.ds(base, epw)])

    return k(*pos_cols, dst_p)


def _segment_max(src_p, pe, ep):
    npw = ep // NW
    half = ep // 2

    @functools.partial(
        pl.kernel,
        mesh=_sc_mesh(),
        compiler_params=pltpu.CompilerParams(needs_layout_passes=False),
        out_type=jax.ShapeDtypeStruct((ep,), jnp.float32),
        scratch_types=[pltpu.VMEM((half,), jnp.int32),
                       pltpu.VMEM((half,), jnp.float32),
                       pltpu.VMEM((16 * npw,), jnp.float32),
                       pltpu.VMEM((npw,), jnp.float32)],
    )
    def k(src_hbm, pe_hbm, seg_hbm, src_v, pe_v, seg16_v, seg_v):
        n0 = _worker_base(npw)
        iota = lax.iota(jnp.int32, 16)
        plane = iota * npw

        def zi(j, carry):
            seg16_v[pl.ds(j * 16, 16)] = jnp.zeros((16,), jnp.float32)
            return carry

        lax.fori_loop(0, npw, zi, 0)

        for hb in range(2):
            pltpu.sync_copy(src_hbm.at[pl.ds(hb * half, half)], src_v)
            pltpu.sync_copy(pe_hbm.at[pl.ds(hb * half, half)], pe_v)

            def chunk(j, carry):
                sl = pl.ds(j * 16, 16)
                loc = src_v[sl] - n0
                v = pe_v[sl]
                m = (loc >= 0) & (loc < npw)
                locc = jnp.clip(loc, 0, npw - 1) + plane
                old = plsc.load_gather(seg16_v, [locc])
                plsc.store_scatter(seg16_v, [locc], jnp.maximum(old, v), mask=m)
                return carry

            lax.fori_loop(0, half // 16, chunk, 0)

        def red(j, carry):
            acc = seg16_v[pl.ds(j * 16, 16)]
            for p in range(1, 16):
                acc = jnp.maximum(acc, seg16_v[pl.ds(p * npw + j * 16, 16)])
            seg_v[pl.ds(j * 16, 16)] = acc
            return carry

        lax.fori_loop(0, npw // 16, red, 0)
        pltpu.sync_copy(seg_v, seg_hbm.at[pl.ds(n0, npw)])

    return k(src_p, pe)


def _edge_assemble(seg, src_p, dst_p, rel, v_nodes, ep):
    epw = ep // NW
    nch = epw // CH
    nb = 4
    h = v_nodes.shape[1]

    @functools.partial(
        pl.kernel,
        mesh=_sc_mesh(),
        compiler_params=pltpu.CompilerParams(needs_layout_passes=False),
        out_type=[jax.ShapeDtypeStruct((4, ep), jnp.float32),
                  jax.ShapeDtypeStruct((ep, h), jnp.float32)],
        scratch_types=[pltpu.VMEM((epw,), jnp.int32),
                       pltpu.VMEM((epw,), jnp.int32),
                       pltpu.VMEM((epw,), jnp.float32),
                       pltpu.VMEM((4, epw), jnp.float32),
                       pltpu.VMEM((4, epw), jnp.float32)]
        + [pltpu.VMEM((CH, h), jnp.float32) for _ in range(nb)]
        + [pltpu.SemaphoreType.DMA,
           pltpu.SemaphoreType.DMA,
           pltpu.SemaphoreType.DMA],
    )
    def k(seg_hbm, src_hbm, dst_hbm, rel_hbm, v_hbm, geom_hbm, vg_hbm,
          sidx_v, didx_v, maxd_v, rel_v, geom_v, *bufs_and_sems):
        bufs = bufs_and_sems[:nb]
        gsem, osem, msem = bufs_and_sems[nb:]
        base = _worker_base(epw)
        pltpu.sync_copy(dst_hbm.at[pl.ds(base, epw)], didx_v)

        gcp = [pltpu.async_copy(v_hbm.at[didx_v.at[pl.ds(kk * CH, CH)]],
                                bufs[kk % nb], gsem)
               for kk in range(nb)]
        gcp += [None] * (nch - nb)

        pltpu.sync_copy(src_hbm.at[pl.ds(base, epw)], sidx_v)
        mcp = [pltpu.async_copy(seg_hbm.at[sidx_v.at[pl.ds(kk * CH, CH)]],
                                maxd_v.at[pl.ds(kk * CH, CH)], msem)
               for kk in range(nch)]
        pltpu.sync_copy(rel_hbm.at[:, pl.ds(base, epw)], rel_v)
        for cp in mcp:
            cp.wait()

        def chunk(j, carry):
            sl = pl.ds(j * 16, 16)
            inv = 1.0 / jnp.maximum(maxd_v[sl], 1e-8)
            geom_v[0, sl] = rel_v[0, sl] * inv
            geom_v[1, sl] = rel_v[1, sl] * inv
            geom_v[2, sl] = rel_v[2, sl] * inv
            geom_v[3, sl] = rel_v[3, sl]
            return carry

        lax.fori_loop(0, epw // 16, chunk, 0)
        pltpu.sync_copy(geom_v, geom_hbm.at[:, pl.ds(base, epw)])

        ocp = {}
        for kk in range(nch):
            gcp[kk].wait()
            ocp[kk] = pltpu.async_copy(
                bufs[kk % nb], vg_hbm.at[pl.ds(base + kk * CH, CH)], osem)
            prev = kk - 1
            if prev >= 0 and prev + nb < nch:
                ocp[prev].wait()
                gcp[prev + nb] = pltpu.async_copy(
                    v_hbm.at[didx_v.at[pl.ds((prev + nb) * CH, CH)]],
                    bufs[prev % nb], gsem)
        for kk in range(max(nch - nb, 0), nch):
            if kk >= 0:
                ocp[kk].wait()

    return k(seg, src_p, dst_p, rel, v_nodes)


def _h_block(vg_ref, g_ref, w1g_ref):
    acc = vg_ref[...]
    acc += lax.dot_general(g_ref[...], w1g_ref[...],
                           (((0,), (0,)), ((), ())),
                           preferred_element_type=jnp.float32)
    return jnp.maximum(acc, 0.0)


def _stats(vg, geom, W1g, e_real):
    ep, hp = vg.shape
    h = hp
    nb_grid = pl.cdiv(e_real, BLK)

    def body(vg_ref, g_ref, w1g_ref, s_ref, q_ref):
        i = pl.program_id(0)
        acc = _h_block(vg_ref, g_ref, w1g_ref)
        rows = i * BLK + lax.broadcasted_iota(jnp.int32, (BLK, 1), 0)
        acc = jnp.where(rows < e_real, acc, 0.0)

        @pl.when(i == 0)
        def _():
            s_ref[...] = jnp.zeros_like(s_ref)
            q_ref[...] = jnp.zeros_like(q_ref)

        s_ref[...] += jnp.sum(acc, axis=0, keepdims=True)
        q_ref[...] += jnp.sum(acc * acc, axis=0, keepdims=True)

    return pl.pallas_call(
        body,
        grid=(nb_grid,),
        in_specs=[
            pl.BlockSpec((BLK, hp), lambda i: (i, 0)),
            pl.BlockSpec((4, BLK), lambda i: (0, i)),
            pl.BlockSpec((4, h), lambda i: (0, 0)),
        ],
        out_specs=[
            pl.BlockSpec((1, h), lambda i: (0, 0)),
            pl.BlockSpec((1, h), lambda i: (0, 0)),
        ],
        out_shape=[jax.ShapeDtypeStruct((1, h), jnp.float32),
                   jax.ShapeDtypeStruct((1, h), jnp.float32)],
    )(vg, geom, W1g)


def _normalize(vg, geom, W1g, s, q, gamma, beta, e_real):
    ep, hp = vg.shape
    h = hp
    nb_grid = pl.cdiv(e_real, BLK)
    inv_n = 1.0 / e_real

    def body(vg_ref, g_ref, w1g_ref, s_ref, q_ref, gam_ref, bet_ref, o_ref):
        acc = _h_block(vg_ref, g_ref, w1g_ref)
        mean = s_ref[...] * inv_n
        var = q_ref[...] * inv_n - mean * mean
        scale = gam_ref[...] / jnp.sqrt(var + 1e-5)
        o_ref[...] = (acc - mean) * scale + bet_ref[...]

    return pl.pallas_call(
        body,
        grid=(nb_grid,),
        in_specs=[
            pl.BlockSpec((BLK, hp), lambda i: (i, 0)),
            pl.BlockSpec((4, BLK), lambda i: (0, i)),
            pl.BlockSpec((4, h), lambda i: (0, 0)),
            pl.BlockSpec((1, h), lambda i: (0, 0)),
            pl.BlockSpec((1, h), lambda i: (0, 0)),
            pl.BlockSpec((1, h), lambda i: (0, 0)),
            pl.BlockSpec((1, h), lambda i: (0, 0)),
        ],
        out_specs=pl.BlockSpec((BLK, h), lambda i: (i, 0)),
        out_shape=jax.ShapeDtypeStruct((e_real, h), jnp.float32),
    )(vg, geom, W1g, s, q, gamma, beta)


def kernel(x, pos, edge_index, Wq, bq, Wk, bk, Wv, bv, W1, b1, gamma, beta):
    n, c = x.shape
    e = edge_index.shape[1]
    h = W1.shape[1]

    src = edge_index[0].astype(jnp.int32)
    dst = edge_index[1].astype(jnp.int32)

    epw = -(-e // (NW * CH)) * CH
    ep = NW * epw
    src_p = jnp.concatenate([src, jnp.zeros((ep - e,), jnp.int32)])
    dst_p = jnp.concatenate([dst, jnp.zeros((ep - e,), jnp.int32)])
    zpad = jnp.zeros((ep - n,), jnp.float32)
    pos_cols = [jnp.concatenate([pos[:, i], zpad]) for i in range(4)]

    W1v = W1[:c]
    W1g = W1[c:]

    v_nodes = _node_values(x, Wv, bv[None, :], W1v, b1[None, :])
    pe, rel = _edge_geometry(pos_cols, dst_p, ep, e)
    seg = _segment_max(src_p, pe, ep)
    geom, vg = _edge_assemble(seg, src_p, dst_p, rel, v_nodes, ep)
    s, q = _stats(vg, geom, W1g, e)
    return _normalize(vg, geom, W1g, s, q, gamma[None, :], beta[None, :], e)


# speedup vs baseline: 2.0872x; 1.2118x over previous
import functools

import jax
import jax.numpy as jnp
from jax import lax
from jax.experimental import pallas as pl
from jax.experimental.pallas import tpu as pltpu
from jax.experimental.pallas import tpu_sc as plsc

NC = 2
NS = 16
NW = NC * NS
CH = 128
BLK = 512
BLKS = 2048


def _node_values(x, Wv, bv, W1v, b1):
    n, c = x.shape
    h = W1v.shape[1]
    nb = pl.cdiv(n, BLK)

    def body(x_ref, wv_ref, bv_ref, w1v_ref, b1_ref, v_ref, weff_ref, beff_ref):
        @pl.when(pl.program_id(0) == 0)
        def _():
            weff_ref[...] = jnp.dot(wv_ref[...], w1v_ref[...],
                                    preferred_element_type=jnp.float32)
            beff_ref[...] = jnp.dot(bv_ref[...], w1v_ref[...],
                                    preferred_element_type=jnp.float32) + b1_ref[...]

        v_ref[...] = jnp.dot(x_ref[...], weff_ref[...],
                             preferred_element_type=jnp.float32) + beff_ref[...]

    return pl.pallas_call(
        body,
        grid=(nb,),
        in_specs=[
            pl.BlockSpec((BLK, c), lambda i: (i, 0)),
            pl.BlockSpec((c, c), lambda i: (0, 0)),
            pl.BlockSpec((1, c), lambda i: (0, 0)),
            pl.BlockSpec((c, h), lambda i: (0, 0)),
            pl.BlockSpec((1, h), lambda i: (0, 0)),
        ],
        out_specs=pl.BlockSpec((BLK, h), lambda i: (i, 0)),
        out_shape=jax.ShapeDtypeStruct((n, h), jnp.float32),
        scratch_shapes=[pltpu.VMEM((c, h), jnp.float32),
                        pltpu.VMEM((1, h), jnp.float32)],
    )(x, Wv, bv, W1v, b1)


def _sc_mesh():
    return plsc.VectorSubcoreMesh(core_axis_name="c", subcore_axis_name="s")


def _worker_base(epw):
    wid = lax.axis_index("s") * NC + lax.axis_index("c")
    return wid * epw


def _edge_geometry(pos_cols, dst_p, ep, e_real):
    epw = ep // NW
    nch = epw // CH

    @functools.partial(
        pl.kernel,
        mesh=_sc_mesh(),
        compiler_params=pltpu.CompilerParams(needs_layout_passes=False),
        out_type=[jax.ShapeDtypeStruct((ep,), jnp.float32),
                  jax.ShapeDtypeStruct((4, ep), jnp.float32)],
        scratch_types=[pltpu.VMEM((epw,), jnp.int32)]
        + [pltpu.VMEM((epw,), jnp.float32) for _ in range(7)]
        + [pltpu.VMEM((epw,), jnp.float32),
           pltpu.VMEM((4, epw), jnp.float32),
           pltpu.SemaphoreType.DMA],
    )
    def k(px_hbm, py_hbm, pz_hbm, pw_hbm, dst_hbm, pe_hbm, rel_hbm,
          idx_v, pjx_v, pjy_v, pjz_v, pjw_v, pix_v, piy_v, piz_v,
          pe_v, rel_v, sem):
        base = _worker_base(epw)
        pltpu.sync_copy(dst_hbm.at[pl.ds(base, epw)], idx_v)
        pltpu.sync_copy(px_hbm.at[pl.ds(base, epw)], pix_v)
        pltpu.sync_copy(py_hbm.at[pl.ds(base, epw)], piy_v)
        pltpu.sync_copy(pz_hbm.at[pl.ds(base, epw)], piz_v)
        cps = []
        for kk in range(nch):
            isl = idx_v.at[pl.ds(kk * CH, CH)]
            osl = pl.ds(kk * CH, CH)
            for tab, dv in ((px_hbm, pjx_v), (py_hbm, pjy_v),
                            (pz_hbm, pjz_v), (pw_hbm, pjw_v)):
                cps.append(pltpu.async_copy(tab.at[isl], dv.at[osl], sem))
        for cp in cps:
            cp.wait()

        def chunk(j, carry):
            r = j * 16
            sl = pl.ds(r, 16)
            rows = lax.iota(jnp.int32, 16) + r
            rx = pjx_v[sl] - pix_v[sl]
            ry = pjy_v[sl] - piy_v[sl]
            rz = pjz_v[sl] - piz_v[sl]
            pe = jnp.maximum(jnp.maximum(jnp.abs(rx), jnp.abs(ry)), jnp.abs(rz))
            pe = jnp.where(rows + base < e_real, pe, 0.0)
            pe_v[sl] = pe
            rel_v[0, sl] = rx
            rel_v[1, sl] = ry
            rel_v[2, sl] = rz
            rel_v[3, sl] = pjw_v[sl]
            return carry

        lax.fori_loop(0, epw // 16, chunk, 0)
        pltpu.sync_copy(pe_v, pe_hbm.at[pl.ds(base, epw)])
        pltpu.sync_copy(rel_v, rel_hbm.at[:, pl.ds(base, epw)])

    return k(*pos_cols, dst_p)


def _segment_max(src_p, pe, ep):
    npw = ep // NW
    half = ep // 2

    @functools.partial(
        pl.kernel,
        mesh=_sc_mesh(),
        compiler_params=pltpu.CompilerParams(needs_layout_passes=False),
        out_type=jax.ShapeDtypeStruct((ep,), jnp.float32),
        scratch_types=[pltpu.VMEM((half,), jnp.int32),
                       pltpu.VMEM((half,), jnp.float32),
                       pltpu.VMEM((16 * npw,), jnp.float32),
                       pltpu.VMEM((npw,), jnp.float32)],
    )
    def k(src_hbm, pe_hbm, seg_hbm, src_v, pe_v, seg16_v, seg_v):
        n0 = _worker_base(npw)
        iota = lax.iota(jnp.int32, 16)
        plane = iota * npw

        def zi(j, carry):
            seg16_v[pl.ds(j * 16, 16)] = jnp.zeros((16,), jnp.float32)
            return carry

        lax.fori_loop(0, npw, zi, 0)

        for hb in range(2):
            pltpu.sync_copy(src_hbm.at[pl.ds(hb * half, half)], src_v)
            pltpu.sync_copy(pe_hbm.at[pl.ds(hb * half, half)], pe_v)

            def chunk(j, carry):
                sl = pl.ds(j * 16, 16)
                loc = src_v[sl] - n0
                v = pe_v[sl]
                m = (loc >= 0) & (loc < npw)
                locc = jnp.clip(loc, 0, npw - 1) + plane
                old = plsc.load_gather(seg16_v, [locc])
                plsc.store_scatter(seg16_v, [locc], jnp.maximum(old, v), mask=m)
                return carry

            lax.fori_loop(0, half // 16, chunk, 0)

        def red(j, carry):
            acc = seg16_v[pl.ds(j * 16, 16)]
            for p in range(1, 16):
                acc = jnp.maximum(acc, seg16_v[pl.ds(p * npw + j * 16, 16)])
            seg_v[pl.ds(j * 16, 16)] = acc
            return carry

        lax.fori_loop(0, npw // 16, red, 0)
        pltpu.sync_copy(seg_v, seg_hbm.at[pl.ds(n0, npw)])

    return k(src_p, pe)


def _edge_assemble(seg, src_p, dst_p, rel, v_nodes, ep):
    epw = ep // NW
    nch = epw // CH
    nb = 4
    h = v_nodes.shape[1]

    @functools.partial(
        pl.kernel,
        mesh=_sc_mesh(),
        compiler_params=pltpu.CompilerParams(needs_layout_passes=False),
        out_type=[jax.ShapeDtypeStruct((4, ep), jnp.float32),
                  jax.ShapeDtypeStruct((ep, h), jnp.float32)],
        scratch_types=[pltpu.VMEM((epw,), jnp.int32),
                       pltpu.VMEM((epw,), jnp.int32),
                       pltpu.VMEM((epw,), jnp.float32),
                       pltpu.VMEM((4, epw), jnp.float32),
                       pltpu.VMEM((4, epw), jnp.float32)]
        + [pltpu.VMEM((CH, h), jnp.float32) for _ in range(nb)]
        + [pltpu.SemaphoreType.DMA,
           pltpu.SemaphoreType.DMA,
           pltpu.SemaphoreType.DMA],
    )
    def k(seg_hbm, src_hbm, dst_hbm, rel_hbm, v_hbm, geom_hbm, vg_hbm,
          sidx_v, didx_v, maxd_v, rel_v, geom_v, *bufs_and_sems):
        bufs = bufs_and_sems[:nb]
        gsem, osem, msem = bufs_and_sems[nb:]
        base = _worker_base(epw)
        pltpu.sync_copy(dst_hbm.at[pl.ds(base, epw)], didx_v)

        gcp = [pltpu.async_copy(v_hbm.at[didx_v.at[pl.ds(kk * CH, CH)]],
                                bufs[kk % nb], gsem)
               for kk in range(nb)]
        gcp += [None] * (nch - nb)

        pltpu.sync_copy(src_hbm.at[pl.ds(base, epw)], sidx_v)
        mcp = [pltpu.async_copy(seg_hbm.at[sidx_v.at[pl.ds(kk * CH, CH)]],
                                maxd_v.at[pl.ds(kk * CH, CH)], msem)
               for kk in range(nch)]
        pltpu.sync_copy(rel_hbm.at[:, pl.ds(base, epw)], rel_v)
        for cp in mcp:
            cp.wait()

        def chunk(j, carry):
            sl = pl.ds(j * 16, 16)
            inv = 1.0 / jnp.maximum(maxd_v[sl], 1e-8)
            geom_v[0, sl] = rel_v[0, sl] * inv
            geom_v[1, sl] = rel_v[1, sl] * inv
            geom_v[2, sl] = rel_v[2, sl] * inv
            geom_v[3, sl] = rel_v[3, sl]
            return carry

        lax.fori_loop(0, epw // 16, chunk, 0)
        pltpu.sync_copy(geom_v, geom_hbm.at[:, pl.ds(base, epw)])

        ocp = {}
        for kk in range(nch):
            gcp[kk].wait()
            ocp[kk] = pltpu.async_copy(
                bufs[kk % nb], vg_hbm.at[pl.ds(base + kk * CH, CH)], osem)
            prev = kk - 1
            if prev >= 0 and prev + nb < nch:
                ocp[prev].wait()
                gcp[prev + nb] = pltpu.async_copy(
                    v_hbm.at[didx_v.at[pl.ds((prev + nb) * CH, CH)]],
                    bufs[prev % nb], gsem)
        for kk in range(max(nch - nb, 0), nch):
            if kk >= 0:
                ocp[kk].wait()

    return k(seg, src_p, dst_p, rel, v_nodes)


def _h_block(vg_ref, g_ref, w1g_ref):
    acc = vg_ref[...]
    acc += lax.dot_general(g_ref[...], w1g_ref[...],
                           (((0,), (0,)), ((), ())),
                           preferred_element_type=jnp.float32)
    return jnp.maximum(acc, 0.0)


def _stats(vg, geom, W1g, e_real):
    ep, hp = vg.shape
    h = hp
    nb_grid = pl.cdiv(e_real, BLKS)

    def body(vg_ref, g_ref, w1g_ref, s_ref, q_ref):
        i = pl.program_id(0)
        acc = _h_block(vg_ref, g_ref, w1g_ref)
        rows = i * BLKS + lax.broadcasted_iota(jnp.int32, (BLKS, 1), 0)
        acc = jnp.where(rows < e_real, acc, 0.0)

        @pl.when(i == 0)
        def _():
            s_ref[...] = jnp.zeros_like(s_ref)
            q_ref[...] = jnp.zeros_like(q_ref)

        s_ref[...] += jnp.sum(acc, axis=0, keepdims=True)
        q_ref[...] += jnp.sum(acc * acc, axis=0, keepdims=True)

    return pl.pallas_call(
        body,
        grid=(nb_grid,),
        in_specs=[
            pl.BlockSpec((BLKS, hp), lambda i: (i, 0)),
            pl.BlockSpec((4, BLKS), lambda i: (0, i)),
            pl.BlockSpec((4, h), lambda i: (0, 0)),
        ],
        out_specs=[
            pl.BlockSpec((1, h), lambda i: (0, 0)),
            pl.BlockSpec((1, h), lambda i: (0, 0)),
        ],
        out_shape=[jax.ShapeDtypeStruct((1, h), jnp.float32),
                   jax.ShapeDtypeStruct((1, h), jnp.float32)],
    )(vg, geom, W1g)


def _normalize(vg, geom, W1g, s, q, gamma, beta, e_real):
    ep, hp = vg.shape
    h = hp
    nb_grid = pl.cdiv(e_real, BLKS)
    inv_n = 1.0 / e_real

    def body(vg_ref, g_ref, w1g_ref, s_ref, q_ref, gam_ref, bet_ref, o_ref):
        acc = _h_block(vg_ref, g_ref, w1g_ref)
        mean = s_ref[...] * inv_n
        var = q_ref[...] * inv_n - mean * mean
        scale = gam_ref[...] / jnp.sqrt(var + 1e-5)
        o_ref[...] = (acc - mean) * scale + bet_ref[...]

    return pl.pallas_call(
        body,
        grid=(nb_grid,),
        in_specs=[
            pl.BlockSpec((BLKS, hp), lambda i: (i, 0)),
            pl.BlockSpec((4, BLKS), lambda i: (0, i)),
            pl.BlockSpec((4, h), lambda i: (0, 0)),
            pl.BlockSpec((1, h), lambda i: (0, 0)),
            pl.BlockSpec((1, h), lambda i: (0, 0)),
            pl.BlockSpec((1, h), lambda i: (0, 0)),
            pl.BlockSpec((1, h), lambda i: (0, 0)),
        ],
        out_specs=pl.BlockSpec((BLKS, h), lambda i: (i, 0)),
        out_shape=jax.ShapeDtypeStruct((e_real, h), jnp.float32),
    )(vg, geom, W1g, s, q, gamma, beta)


def kernel(x, pos, edge_index, Wq, bq, Wk, bk, Wv, bv, W1, b1, gamma, beta):
    n, c = x.shape
    e = edge_index.shape[1]
    h = W1.shape[1]

    src = edge_index[0].astype(jnp.int32)
    dst = edge_index[1].astype(jnp.int32)

    epw = -(-e // (NW * CH)) * CH
    ep = NW * epw
    src_p = jnp.concatenate([src, jnp.zeros((ep - e,), jnp.int32)])
    dst_p = jnp.concatenate([dst, jnp.zeros((ep - e,), jnp.int32)])
    zpad = jnp.zeros((ep - n,), jnp.float32)
    pos_cols = [jnp.concatenate([pos[:, i], zpad]) for i in range(4)]

    W1v = W1[:c]
    W1g = W1[c:]

    v_nodes = _node_values(x, Wv, bv[None, :], W1v, b1[None, :])
    pe, rel = _edge_geometry(pos_cols, dst_p, ep, e)
    seg = _segment_max(src_p, pe, ep)
    geom, vg = _edge_assemble(seg, src_p, dst_p, rel, v_nodes, ep)
    s, q = _stats(vg, geom, W1g, e)
    return _normalize(vg, geom, W1g, s, q, gamma[None, :], beta[None, :], e)


# speedup vs baseline: 2.0940x; 1.0033x over previous
import functools

import jax
import jax.numpy as jnp
from jax import lax
from jax.experimental import pallas as pl
from jax.experimental.pallas import tpu as pltpu
from jax.experimental.pallas import tpu_sc as plsc

NC = 2
NS = 16
NW = NC * NS
CH = 128
BLK = 512
BLKS = 2048


def _node_values(x, Wv, bv, W1v, b1):
    n, c = x.shape
    h = W1v.shape[1]
    nb = pl.cdiv(n, BLK)

    def body(x_ref, wv_ref, bv_ref, w1v_ref, b1_ref, v_ref, weff_ref, beff_ref):
        @pl.when(pl.program_id(0) == 0)
        def _():
            weff_ref[...] = jnp.dot(wv_ref[...], w1v_ref[...],
                                    preferred_element_type=jnp.float32)
            beff_ref[...] = jnp.dot(bv_ref[...], w1v_ref[...],
                                    preferred_element_type=jnp.float32) + b1_ref[...]

        v_ref[...] = jnp.dot(x_ref[...], weff_ref[...],
                             preferred_element_type=jnp.float32) + beff_ref[...]

    return pl.pallas_call(
        body,
        grid=(nb,),
        in_specs=[
            pl.BlockSpec((BLK, c), lambda i: (i, 0)),
            pl.BlockSpec((c, c), lambda i: (0, 0)),
            pl.BlockSpec((1, c), lambda i: (0, 0)),
            pl.BlockSpec((c, h), lambda i: (0, 0)),
            pl.BlockSpec((1, h), lambda i: (0, 0)),
        ],
        out_specs=pl.BlockSpec((BLK, h), lambda i: (i, 0)),
        out_shape=jax.ShapeDtypeStruct((n, h), jnp.float32),
        scratch_shapes=[pltpu.VMEM((c, h), jnp.float32),
                        pltpu.VMEM((1, h), jnp.float32)],
    )(x, Wv, bv, W1v, b1)


def _sc_mesh():
    return plsc.VectorSubcoreMesh(core_axis_name="c", subcore_axis_name="s")


def _worker_base(epw):
    wid = lax.axis_index("s") * NC + lax.axis_index("c")
    return wid * epw


def _edge_geometry(pos_cols, dst_p, ep, e_real):
    epw = ep // NW
    nch = epw // CH

    @functools.partial(
        pl.kernel,
        mesh=_sc_mesh(),
        compiler_params=pltpu.CompilerParams(needs_layout_passes=False),
        out_type=[jax.ShapeDtypeStruct((ep,), jnp.float32),
                  jax.ShapeDtypeStruct((4, ep), jnp.float32)],
        scratch_types=[pltpu.VMEM((epw,), jnp.int32)]
        + [pltpu.VMEM((epw,), jnp.float32) for _ in range(7)]
        + [pltpu.VMEM((epw,), jnp.float32),
           pltpu.VMEM((4, epw), jnp.float32),
           pltpu.SemaphoreType.DMA],
    )
    def k(px_hbm, py_hbm, pz_hbm, pw_hbm, dst_hbm, pe_hbm, rel_hbm,
          idx_v, pjx_v, pjy_v, pjz_v, pjw_v, pix_v, piy_v, piz_v,
          pe_v, rel_v, sem):
        base = _worker_base(epw)
        pltpu.sync_copy(dst_hbm.at[pl.ds(base, epw)], idx_v)
        pltpu.sync_copy(px_hbm.at[pl.ds(base, epw)], pix_v)
        pltpu.sync_copy(py_hbm.at[pl.ds(base, epw)], piy_v)
        pltpu.sync_copy(pz_hbm.at[pl.ds(base, epw)], piz_v)
        cps = []
        for kk in range(nch):
            isl = idx_v.at[pl.ds(kk * CH, CH)]
            osl = pl.ds(kk * CH, CH)
            for tab, dv in ((px_hbm, pjx_v), (py_hbm, pjy_v),
                            (pz_hbm, pjz_v), (pw_hbm, pjw_v)):
                cps.append(pltpu.async_copy(tab.at[isl], dv.at[osl], sem))
        for cp in cps:
            cp.wait()

        def chunk(j, carry):
            r = j * 16
            sl = pl.ds(r, 16)
            rows = lax.iota(jnp.int32, 16) + r
            rx = pjx_v[sl] - pix_v[sl]
            ry = pjy_v[sl] - piy_v[sl]
            rz = pjz_v[sl] - piz_v[sl]
            pe = jnp.maximum(jnp.maximum(jnp.abs(rx), jnp.abs(ry)), jnp.abs(rz))
            pe = jnp.where(rows + base < e_real, pe, 0.0)
            pe_v[sl] = pe
            rel_v[0, sl] = rx
            rel_v[1, sl] = ry
            rel_v[2, sl] = rz
            rel_v[3, sl] = pjw_v[sl]
            return carry

        lax.fori_loop(0, epw // 16, chunk, 0)
        pltpu.sync_copy(pe_v, pe_hbm.at[pl.ds(base, epw)])
        pltpu.sync_copy(rel_v, rel_hbm.at[:, pl.ds(base, epw)])

    return k(*pos_cols, dst_p)


def _segment_max(src_p, pe, ep):
    npw = ep // NW
    half = ep // 2

    @functools.partial(
        pl.kernel,
        mesh=_sc_mesh(),
        compiler_params=pltpu.CompilerParams(needs_layout_passes=False),
        out_type=jax.ShapeDtypeStruct((ep,), jnp.float32),
        scratch_types=[pltpu.VMEM((half,), jnp.int32),
                       pltpu.VMEM((half,), jnp.float32),
                       pltpu.VMEM((16 * npw,), jnp.float32),
                       pltpu.VMEM((npw,), jnp.float32)],
    )
    def k(src_hbm, pe_hbm, seg_hbm, src_v, pe_v, seg16_v, seg_v):
        n0 = _worker_base(npw)
        iota = lax.iota(jnp.int32, 16)
        plane = iota * npw

        def zi(j, carry):
            seg16_v[pl.ds(j * 16, 16)] = jnp.zeros((16,), jnp.float32)
            return carry

        lax.fori_loop(0, npw, zi, 0)

        for hb in range(2):
            pltpu.sync_copy(src_hbm.at[pl.ds(hb * half, half)], src_v)
            pltpu.sync_copy(pe_hbm.at[pl.ds(hb * half, half)], pe_v)

            def chunk(j, carry):
                sl = pl.ds(j * 16, 16)
                loc = src_v[sl] - n0
                v = pe_v[sl]
                m = (loc >= 0) & (loc < npw)
                locc = jnp.clip(loc, 0, npw - 1) + plane
                old = plsc.load_gather(seg16_v, [locc])
                plsc.store_scatter(seg16_v, [locc], jnp.maximum(old, v), mask=m)
                return carry

            lax.fori_loop(0, half // 16, chunk, 0)

        def red(j, carry):
            acc = seg16_v[pl.ds(j * 16, 16)]
            for p in range(1, 16):
                acc = jnp.maximum(acc, seg16_v[pl.ds(p * npw + j * 16, 16)])
            seg_v[pl.ds(j * 16, 16)] = acc
            return carry

        lax.fori_loop(0, npw // 16, red, 0)
        pltpu.sync_copy(seg_v, seg_hbm.at[pl.ds(n0, npw)])

    return k(src_p, pe)


def _edge_assemble(seg, src_p, dst_p, rel, v_nodes, ep):
    epw = ep // NW
    nch = epw // CH
    nb = 6
    h = v_nodes.shape[1]

    @functools.partial(
        pl.kernel,
        mesh=_sc_mesh(),
        compiler_params=pltpu.CompilerParams(needs_layout_passes=False),
        out_type=[jax.ShapeDtypeStruct((4, ep), jnp.float32),
                  jax.ShapeDtypeStruct((ep, h), jnp.float32)],
        scratch_types=[pltpu.VMEM((epw,), jnp.int32),
                       pltpu.VMEM((epw,), jnp.int32),
                       pltpu.VMEM((epw,), jnp.float32),
                       pltpu.VMEM((4, epw), jnp.float32),
                       pltpu.VMEM((4, epw), jnp.float32)]
        + [pltpu.VMEM((CH, h), jnp.float32) for _ in range(nb)]
        + [pltpu.SemaphoreType.DMA,
           pltpu.SemaphoreType.DMA,
           pltpu.SemaphoreType.DMA],
    )
    def k(seg_hbm, src_hbm, dst_hbm, rel_hbm, v_hbm, geom_hbm, vg_hbm,
          sidx_v, didx_v, maxd_v, rel_v, geom_v, *bufs_and_sems):
        bufs = bufs_and_sems[:nb]
        gsem, osem, msem = bufs_and_sems[nb:]
        base = _worker_base(epw)
        pltpu.sync_copy(dst_hbm.at[pl.ds(base, epw)], didx_v)

        gcp = [pltpu.async_copy(v_hbm.at[didx_v.at[pl.ds(kk * CH, CH)]],
                                bufs[kk % nb], gsem)
               for kk in range(nb)]
        gcp += [None] * (nch - nb)

        pltpu.sync_copy(src_hbm.at[pl.ds(base, epw)], sidx_v)
        mcp = [pltpu.async_copy(seg_hbm.at[sidx_v.at[pl.ds(kk * CH, CH)]],
                                maxd_v.at[pl.ds(kk * CH, CH)], msem)
               for kk in range(nch)]
        pltpu.sync_copy(rel_hbm.at[:, pl.ds(base, epw)], rel_v)
        for cp in mcp:
            cp.wait()

        def chunk(j, carry):
            sl = pl.ds(j * 16, 16)
            inv = 1.0 / jnp.maximum(maxd_v[sl], 1e-8)
            geom_v[0, sl] = rel_v[0, sl] * inv
            geom_v[1, sl] = rel_v[1, sl] * inv
            geom_v[2, sl] = rel_v[2, sl] * inv
            geom_v[3, sl] = rel_v[3, sl]
            return carry

        lax.fori_loop(0, epw // 16, chunk, 0)
        pltpu.sync_copy(geom_v, geom_hbm.at[:, pl.ds(base, epw)])

        ocp = {}
        for kk in range(nch):
            gcp[kk].wait()
            ocp[kk] = pltpu.async_copy(
                bufs[kk % nb], vg_hbm.at[pl.ds(base + kk * CH, CH)], osem)
            prev = kk - 1
            if prev >= 0 and prev + nb < nch:
                ocp[prev].wait()
                gcp[prev + nb] = pltpu.async_copy(
                    v_hbm.at[didx_v.at[pl.ds((prev + nb) * CH, CH)]],
                    bufs[prev % nb], gsem)
        for kk in range(max(nch - nb, 0), nch):
            if kk >= 0:
                ocp[kk].wait()

    return k(seg, src_p, dst_p, rel, v_nodes)


def _h_block(vg_ref, g_ref, w1g_ref):
    acc = vg_ref[...]
    acc += lax.dot_general(g_ref[...], w1g_ref[...],
                           (((0,), (0,)), ((), ())),
                           preferred_element_type=jnp.float32)
    return jnp.maximum(acc, 0.0)


def _stats(vg, geom, W1g, e_real):
    ep, hp = vg.shape
    h = hp
    nb_grid = pl.cdiv(e_real, BLKS)

    def body(vg_ref, g_ref, w1g_ref, s_ref, q_ref):
        i = pl.program_id(0)
        acc = _h_block(vg_ref, g_ref, w1g_ref)
        rows = i * BLKS + lax.broadcasted_iota(jnp.int32, (BLKS, 1), 0)
        acc = jnp.where(rows < e_real, acc, 0.0)

        @pl.when(i == 0)
        def _():
            s_ref[...] = jnp.zeros_like(s_ref)
            q_ref[...] = jnp.zeros_like(q_ref)

        s_ref[...] += jnp.sum(acc, axis=0, keepdims=True)
        q_ref[...] += jnp.sum(acc * acc, axis=0, keepdims=True)

    return pl.pallas_call(
        body,
        grid=(nb_grid,),
        in_specs=[
            pl.BlockSpec((BLKS, hp), lambda i: (i, 0)),
            pl.BlockSpec((4, BLKS), lambda i: (0, i)),
            pl.BlockSpec((4, h), lambda i: (0, 0)),
        ],
        out_specs=[
            pl.BlockSpec((1, h), lambda i: (0, 0)),
            pl.BlockSpec((1, h), lambda i: (0, 0)),
        ],
        out_shape=[jax.ShapeDtypeStruct((1, h), jnp.float32),
                   jax.ShapeDtypeStruct((1, h), jnp.float32)],
    )(vg, geom, W1g)


def _normalize(vg, geom, W1g, s, q, gamma, beta, e_real):
    ep, hp = vg.shape
    h = hp
    nb_grid = pl.cdiv(e_real, BLKS)
    inv_n = 1.0 / e_real

    def body(vg_ref, g_ref, w1g_ref, s_ref, q_ref, gam_ref, bet_ref, o_ref):
        acc = _h_block(vg_ref, g_ref, w1g_ref)
        mean = s_ref[...] * inv_n
        var = q_ref[...] * inv_n - mean * mean
        scale = gam_ref[...] / jnp.sqrt(var + 1e-5)
        o_ref[...] = (acc - mean) * scale + bet_ref[...]

    return pl.pallas_call(
        body,
        grid=(nb_grid,),
        in_specs=[
            pl.BlockSpec((BLKS, hp), lambda i: (i, 0)),
            pl.BlockSpec((4, BLKS), lambda i: (0, i)),
            pl.BlockSpec((4, h), lambda i: (0, 0)),
            pl.BlockSpec((1, h), lambda i: (0, 0)),
            pl.BlockSpec((1, h), lambda i: (0, 0)),
            pl.BlockSpec((1, h), lambda i: (0, 0)),
            pl.BlockSpec((1, h), lambda i: (0, 0)),
        ],
        out_specs=pl.BlockSpec((BLKS, h), lambda i: (i, 0)),
        out_shape=jax.ShapeDtypeStruct((e_real, h), jnp.float32),
    )(vg, geom, W1g, s, q, gamma, beta)


def kernel(x, pos, edge_index, Wq, bq, Wk, bk, Wv, bv, W1, b1, gamma, beta):
    n, c = x.shape
    e = edge_index.shape[1]
    h = W1.shape[1]

    src = edge_index[0].astype(jnp.int32)
    dst = edge_index[1].astype(jnp.int32)

    epw = -(-e // (NW * CH)) * CH
    ep = NW * epw
    src_p = jnp.concatenate([src, jnp.zeros((ep - e,), jnp.int32)])
    dst_p = jnp.concatenate([dst, jnp.zeros((ep - e,), jnp.int32)])
    zpad = jnp.zeros((ep - n,), jnp.float32)
    pos_cols = [jnp.concatenate([pos[:, i], zpad]) for i in range(4)]

    W1v = W1[:c]
    W1g = W1[c:]

    v_nodes = _node_values(x, Wv, bv[None, :], W1v, b1[None, :])
    pe, rel = _edge_geometry(pos_cols, dst_p, ep, e)
    seg = _segment_max(src_p, pe, ep)
    geom, vg = _edge_assemble(seg, src_p, dst_p, rel, v_nodes, ep)
    s, q = _stats(vg, geom, W1g, e)
    return _normalize(vg, geom, W1g, s, q, gamma[None, :], beta[None, :], e)


# speedup vs baseline: 2.2452x; 1.0722x over previous
import functools

import jax
import jax.numpy as jnp
from jax import lax
from jax.experimental import pallas as pl
from jax.experimental.pallas import tpu as pltpu
from jax.experimental.pallas import tpu_sc as plsc

NC = 2
NS = 16
NW = NC * NS
CH = 128
BLK = 1024
BLKS = 2048


def _node_values(x, Wv, bv, W1v, b1):
    n, c = x.shape
    h = W1v.shape[1]
    nb = pl.cdiv(n, BLK)

    def body(x_ref, wv_ref, bv_ref, w1v_ref, b1_ref, v_ref, weff_ref, beff_ref):
        @pl.when(pl.program_id(0) == 0)
        def _():
            weff_ref[...] = jnp.dot(wv_ref[...], w1v_ref[...],
                                    preferred_element_type=jnp.float32)
            beff_ref[...] = jnp.dot(bv_ref[...], w1v_ref[...],
                                    preferred_element_type=jnp.float32) + b1_ref[...]

        v_ref[...] = jnp.dot(x_ref[...], weff_ref[...],
                             preferred_element_type=jnp.float32) + beff_ref[...]

    return pl.pallas_call(
        body,
        grid=(nb,),
        in_specs=[
            pl.BlockSpec((BLK, c), lambda i: (i, 0)),
            pl.BlockSpec((c, c), lambda i: (0, 0)),
            pl.BlockSpec((1, c), lambda i: (0, 0)),
            pl.BlockSpec((c, h), lambda i: (0, 0)),
            pl.BlockSpec((1, h), lambda i: (0, 0)),
        ],
        out_specs=pl.BlockSpec((BLK, h), lambda i: (i, 0)),
        out_shape=jax.ShapeDtypeStruct((n, h), jnp.float32),
        scratch_shapes=[pltpu.VMEM((c, h), jnp.float32),
                        pltpu.VMEM((1, h), jnp.float32)],
    )(x, Wv, bv, W1v, b1)


def _sc_mesh():
    return plsc.VectorSubcoreMesh(core_axis_name="c", subcore_axis_name="s")


def _worker_base(epw):
    wid = lax.axis_index("s") * NC + lax.axis_index("c")
    return wid * epw


def _edge_geometry(pos_cols, dst_p, ep, e_real):
    epw = ep // NW
    nch = epw // CH

    @functools.partial(
        pl.kernel,
        mesh=_sc_mesh(),
        compiler_params=pltpu.CompilerParams(needs_layout_passes=False),
        out_type=[jax.ShapeDtypeStruct((ep,), jnp.float32),
                  jax.ShapeDtypeStruct((4, ep), jnp.float32)],
        scratch_types=[pltpu.VMEM((epw,), jnp.int32)]
        + [pltpu.VMEM((epw,), jnp.float32) for _ in range(7)]
        + [pltpu.VMEM((epw,), jnp.float32),
           pltpu.VMEM((4, epw), jnp.float32),
           pltpu.SemaphoreType.DMA],
    )
    def k(px_hbm, py_hbm, pz_hbm, pw_hbm, dst_hbm, pe_hbm, rel_hbm,
          idx_v, pjx_v, pjy_v, pjz_v, pjw_v, pix_v, piy_v, piz_v,
          pe_v, rel_v, sem):
        base = _worker_base(epw)
        pltpu.sync_copy(dst_hbm.at[pl.ds(base, epw)], idx_v)
        pltpu.sync_copy(px_hbm.at[pl.ds(base, epw)], pix_v)
        pltpu.sync_copy(py_hbm.at[pl.ds(base, epw)], piy_v)
        pltpu.sync_copy(pz_hbm.at[pl.ds(base, epw)], piz_v)
        cps = []
        for kk in range(nch):
            isl = idx_v.at[pl.ds(kk * CH, CH)]
            osl = pl.ds(kk * CH, CH)
            for tab, dv in ((px_hbm, pjx_v), (py_hbm, pjy_v),
                            (pz_hbm, pjz_v), (pw_hbm, pjw_v)):
                cps.append(pltpu.async_copy(tab.at[isl], dv.at[osl], sem))
        for cp in cps:
            cp.wait()

        def chunk(j, carry):
            r = j * 16
            sl = pl.ds(r, 16)
            rows = lax.iota(jnp.int32, 16) + r
            rx = pjx_v[sl] - pix_v[sl]
            ry = pjy_v[sl] - piy_v[sl]
            rz = pjz_v[sl] - piz_v[sl]
            pe = jnp.maximum(jnp.maximum(jnp.abs(rx), jnp.abs(ry)), jnp.abs(rz))
            pe = jnp.where(rows + base < e_real, pe, 0.0)
            pe_v[sl] = pe
            rel_v[0, sl] = rx
            rel_v[1, sl] = ry
            rel_v[2, sl] = rz
            rel_v[3, sl] = pjw_v[sl]
            return carry

        lax.fori_loop(0, epw // 16, chunk, 0)
        pltpu.sync_copy(pe_v, pe_hbm.at[pl.ds(base, epw)])
        pltpu.sync_copy(rel_v, rel_hbm.at[:, pl.ds(base, epw)])

    return k(*pos_cols, dst_p)


def _segment_max(src_p, pe, ep):
    npw = ep // NW
    qn = 4
    qs = ep // qn

    @functools.partial(
        pl.kernel,
        mesh=_sc_mesh(),
        compiler_params=pltpu.CompilerParams(needs_layout_passes=False),
        out_type=jax.ShapeDtypeStruct((ep,), jnp.float32),
        scratch_types=[pltpu.VMEM((qs,), jnp.int32),
                       pltpu.VMEM((qs,), jnp.int32),
                       pltpu.VMEM((qs,), jnp.float32),
                       pltpu.VMEM((qs,), jnp.float32),
                       pltpu.VMEM((32 * npw,), jnp.float32),
                       pltpu.VMEM((npw,), jnp.float32),
                       pltpu.SemaphoreType.DMA],
    )
    def k(src_hbm, pe_hbm, seg_hbm, src0_v, src1_v, pe0_v, pe1_v,
          seg32_v, seg_v, sem):
        n0 = _worker_base(npw)
        iota = lax.iota(jnp.int32, 16)
        plane_a = iota * npw
        plane_b = plane_a + 16 * npw
        srcs = (src0_v, src1_v)
        pes = (pe0_v, pe1_v)

        def zi(j, carry):
            seg32_v[pl.ds(j * 16, 16)] = jnp.zeros((16,), jnp.float32)
            return carry

        cps = {0: (pltpu.async_copy(src_hbm.at[pl.ds(0, qs)], src0_v, sem),
                   pltpu.async_copy(pe_hbm.at[pl.ds(0, qs)], pe0_v, sem))}
        lax.fori_loop(0, 2 * npw, zi, 0)

        def scat(sv, pv):
            def chunk(j, carry):
                sla = pl.ds(j * 32, 16)
                slb = pl.ds(j * 32 + 16, 16)
                la = sv[sla] - n0
                lb = sv[slb] - n0
                va = pv[sla]
                vb = pv[slb]
                ma = (la >= 0) & (la < npw)
                mb = (lb >= 0) & (lb < npw)
                ia = jnp.clip(la, 0, npw - 1) + plane_a
                ib = jnp.clip(lb, 0, npw - 1) + plane_b
                oa = plsc.load_gather(seg32_v, [ia])
                ob = plsc.load_gather(seg32_v, [ib])
                plsc.store_scatter(seg32_v, [ia], jnp.maximum(oa, va), mask=ma)
                plsc.store_scatter(seg32_v, [ib], jnp.maximum(ob, vb), mask=mb)
                return carry

            lax.fori_loop(0, qs // 32, chunk, 0)

        for q in range(qn):
            if q + 1 < qn:
                b = (q + 1) % 2
                cps[q + 1] = (
                    pltpu.async_copy(src_hbm.at[pl.ds((q + 1) * qs, qs)],
                                     srcs[b], sem),
                    pltpu.async_copy(pe_hbm.at[pl.ds((q + 1) * qs, qs)],
                                     pes[b], sem))
            cps[q][0].wait()
            cps[q][1].wait()
            scat(srcs[q % 2], pes[q % 2])

        def red(j, carry):
            acc = seg32_v[pl.ds(j * 16, 16)]
            for p in range(1, 32):
                acc = jnp.maximum(acc, seg32_v[pl.ds(p * npw + j * 16, 16)])
            seg_v[pl.ds(j * 16, 16)] = acc
            return carry

        lax.fori_loop(0, npw // 16, red, 0)
        pltpu.sync_copy(seg_v, seg_hbm.at[pl.ds(n0, npw)])

    return k(src_p, pe)


def _edge_assemble(seg, src_p, dst_p, rel, v_nodes, ep):
    epw = ep // NW
    nch = epw // CH
    nb = 6
    h = v_nodes.shape[1]

    @functools.partial(
        pl.kernel,
        mesh=_sc_mesh(),
        compiler_params=pltpu.CompilerParams(needs_layout_passes=False),
        out_type=[jax.ShapeDtypeStruct((4, ep), jnp.float32),
                  jax.ShapeDtypeStruct((ep, h), jnp.float32)],
        scratch_types=[pltpu.VMEM((epw,), jnp.int32),
                       pltpu.VMEM((epw,), jnp.int32),
                       pltpu.VMEM((epw,), jnp.float32),
                       pltpu.VMEM((4, epw), jnp.float32),
                       pltpu.VMEM((4, epw), jnp.float32)]
        + [pltpu.VMEM((CH, h), jnp.float32) for _ in range(nb)]
        + [pltpu.SemaphoreType.DMA,
           pltpu.SemaphoreType.DMA,
           pltpu.SemaphoreType.DMA],
    )
    def k(seg_hbm, src_hbm, dst_hbm, rel_hbm, v_hbm, geom_hbm, vg_hbm,
          sidx_v, didx_v, maxd_v, rel_v, geom_v, *bufs_and_sems):
        bufs = bufs_and_sems[:nb]
        gsem, osem, msem = bufs_and_sems[nb:]
        base = _worker_base(epw)
        pltpu.sync_copy(dst_hbm.at[pl.ds(base, epw)], didx_v)

        gcp = [pltpu.async_copy(v_hbm.at[didx_v.at[pl.ds(kk * CH, CH)]],
                                bufs[kk % nb], gsem)
               for kk in range(nb)]
        gcp += [None] * (nch - nb)

        pltpu.sync_copy(src_hbm.at[pl.ds(base, epw)], sidx_v)
        mcp = [pltpu.async_copy(seg_hbm.at[sidx_v.at[pl.ds(kk * CH, CH)]],
                                maxd_v.at[pl.ds(kk * CH, CH)], msem)
               for kk in range(nch)]
        pltpu.sync_copy(rel_hbm.at[:, pl.ds(base, epw)], rel_v)
        for cp in mcp:
            cp.wait()

        def chunk(j, carry):
            sl = pl.ds(j * 16, 16)
            inv = 1.0 / jnp.maximum(maxd_v[sl], 1e-8)
            geom_v[0, sl] = rel_v[0, sl] * inv
            geom_v[1, sl] = rel_v[1, sl] * inv
            geom_v[2, sl] = rel_v[2, sl] * inv
            geom_v[3, sl] = rel_v[3, sl]
            return carry

        lax.fori_loop(0, epw // 16, chunk, 0)
        pltpu.sync_copy(geom_v, geom_hbm.at[:, pl.ds(base, epw)])

        ocp = {}
        for kk in range(nch):
            gcp[kk].wait()
            ocp[kk] = pltpu.async_copy(
                bufs[kk % nb], vg_hbm.at[pl.ds(base + kk * CH, CH)], osem)
            prev = kk - 1
            if prev >= 0 and prev + nb < nch:
                ocp[prev].wait()
                gcp[prev + nb] = pltpu.async_copy(
                    v_hbm.at[didx_v.at[pl.ds((prev + nb) * CH, CH)]],
                    bufs[prev % nb], gsem)
        for kk in range(max(nch - nb, 0), nch):
            if kk >= 0:
                ocp[kk].wait()

    return k(seg, src_p, dst_p, rel, v_nodes)


def _h_block(vg_ref, g_ref, w1g_ref):
    acc = vg_ref[...]
    acc += lax.dot_general(g_ref[...], w1g_ref[...],
                           (((0,), (0,)), ((), ())),
                           preferred_element_type=jnp.float32)
    return jnp.maximum(acc, 0.0)


def _stats(vg, geom, W1g, e_real):
    ep, hp = vg.shape
    h = hp
    nb_grid = pl.cdiv(e_real, BLKS)

    def body(vg_ref, g_ref, w1g_ref, s_ref, q_ref):
        i = pl.program_id(0)
        acc = _h_block(vg_ref, g_ref, w1g_ref)
        rows = i * BLKS + lax.broadcasted_iota(jnp.int32, (BLKS, 1), 0)
        acc = jnp.where(rows < e_real, acc, 0.0)

        @pl.when(i == 0)
        def _():
            s_ref[...] = jnp.zeros_like(s_ref)
            q_ref[...] = jnp.zeros_like(q_ref)

        s_ref[...] += jnp.sum(acc, axis=0, keepdims=True)
        q_ref[...] += jnp.sum(acc * acc, axis=0, keepdims=True)

    return pl.pallas_call(
        body,
        grid=(nb_grid,),
        in_specs=[
            pl.BlockSpec((BLKS, hp), lambda i: (i, 0)),
            pl.BlockSpec((4, BLKS), lambda i: (0, i)),
            pl.BlockSpec((4, h), lambda i: (0, 0)),
        ],
        out_specs=[
            pl.BlockSpec((1, h), lambda i: (0, 0)),
            pl.BlockSpec((1, h), lambda i: (0, 0)),
        ],
        out_shape=[jax.ShapeDtypeStruct((1, h), jnp.float32),
                   jax.ShapeDtypeStruct((1, h), jnp.float32)],
    )(vg, geom, W1g)


def _normalize(vg, geom, W1g, s, q, gamma, beta, e_real):
    ep, hp = vg.shape
    h = hp
    nb_grid = pl.cdiv(e_real, BLKS)
    inv_n = 1.0 / e_real

    def body(vg_ref, g_ref, w1g_ref, s_ref, q_ref, gam_ref, bet_ref, o_ref):
        acc = _h_block(vg_ref, g_ref, w1g_ref)
        mean = s_ref[...] * inv_n
        var = q_ref[...] * inv_n - mean * mean
        scale = gam_ref[...] / jnp.sqrt(var + 1e-5)
        o_ref[...] = (acc - mean) * scale + bet_ref[...]

    return pl.pallas_call(
        body,
        grid=(nb_grid,),
        in_specs=[
            pl.BlockSpec((BLKS, hp), lambda i: (i, 0)),
            pl.BlockSpec((4, BLKS), lambda i: (0, i)),
            pl.BlockSpec((4, h), lambda i: (0, 0)),
            pl.BlockSpec((1, h), lambda i: (0, 0)),
            pl.BlockSpec((1, h), lambda i: (0, 0)),
            pl.BlockSpec((1, h), lambda i: (0, 0)),
            pl.BlockSpec((1, h), lambda i: (0, 0)),
        ],
        out_specs=pl.BlockSpec((BLKS, h), lambda i: (i, 0)),
        out_shape=jax.ShapeDtypeStruct((e_real, h), jnp.float32),
    )(vg, geom, W1g, s, q, gamma, beta)


def kernel(x, pos, edge_index, Wq, bq, Wk, bk, Wv, bv, W1, b1, gamma, beta):
    n, c = x.shape
    e = edge_index.shape[1]
    h = W1.shape[1]

    src = edge_index[0].astype(jnp.int32)
    dst = edge_index[1].astype(jnp.int32)

    epw = -(-e // (NW * CH)) * CH
    ep = NW * epw
    src_p = jnp.concatenate([src, jnp.zeros((ep - e,), jnp.int32)])
    dst_p = jnp.concatenate([dst, jnp.zeros((ep - e,), jnp.int32)])
    zpad = jnp.zeros((ep - n,), jnp.float32)
    pos_cols = [jnp.concatenate([pos[:, i], zpad]) for i in range(4)]

    W1v = W1[:c]
    W1g = W1[c:]

    v_nodes = _node_values(x, Wv, bv[None, :], W1v, b1[None, :])
    pe, rel = _edge_geometry(pos_cols, dst_p, ep, e)
    seg = _segment_max(src_p, pe, ep)
    geom, vg = _edge_assemble(seg, src_p, dst_p, rel, v_nodes, ep)
    s, q = _stats(vg, geom, W1g, e)
    return _normalize(vg, geom, W1g, s, q, gamma[None, :], beta[None, :], e)


# speedup vs baseline: 2.2494x; 1.0018x over previous
import functools

import jax
import jax.numpy as jnp
from jax import lax
from jax.experimental import pallas as pl
from jax.experimental.pallas import tpu as pltpu
from jax.experimental.pallas import tpu_sc as plsc

NC = 2
NS = 16
NW = NC * NS
CH = 104
BLK = 1024
BLKS = 2048


def _node_values(x, Wv, bv, W1v, b1):
    n, c = x.shape
    h = W1v.shape[1]
    nb = pl.cdiv(n, BLK)

    def body(x_ref, wv_ref, bv_ref, w1v_ref, b1_ref, v_ref, weff_ref, beff_ref):
        @pl.when(pl.program_id(0) == 0)
        def _():
            weff_ref[...] = jnp.dot(wv_ref[...], w1v_ref[...],
                                    preferred_element_type=jnp.float32)
            beff_ref[...] = jnp.dot(bv_ref[...], w1v_ref[...],
                                    preferred_element_type=jnp.float32) + b1_ref[...]

        v_ref[...] = jnp.dot(x_ref[...], weff_ref[...],
                             preferred_element_type=jnp.float32) + beff_ref[...]

    return pl.pallas_call(
        body,
        grid=(nb,),
        in_specs=[
            pl.BlockSpec((BLK, c), lambda i: (i, 0)),
            pl.BlockSpec((c, c), lambda i: (0, 0)),
            pl.BlockSpec((1, c), lambda i: (0, 0)),
            pl.BlockSpec((c, h), lambda i: (0, 0)),
            pl.BlockSpec((1, h), lambda i: (0, 0)),
        ],
        out_specs=pl.BlockSpec((BLK, h), lambda i: (i, 0)),
        out_shape=jax.ShapeDtypeStruct((n, h), jnp.float32),
        scratch_shapes=[pltpu.VMEM((c, h), jnp.float32),
                        pltpu.VMEM((1, h), jnp.float32)],
    )(x, Wv, bv, W1v, b1)


def _sc_mesh():
    return plsc.VectorSubcoreMesh(core_axis_name="c", subcore_axis_name="s")


def _worker_base(epw):
    wid = lax.axis_index("s") * NC + lax.axis_index("c")
    return wid * epw


def _edge_geometry(pos_cols, dst_p, ep, e_real):
    epw = ep // NW
    nch = epw // CH

    @functools.partial(
        pl.kernel,
        mesh=_sc_mesh(),
        compiler_params=pltpu.CompilerParams(needs_layout_passes=False),
        out_type=[jax.ShapeDtypeStruct((ep,), jnp.float32),
                  jax.ShapeDtypeStruct((4, ep), jnp.float32)],
        scratch_types=[pltpu.VMEM((epw,), jnp.int32)]
        + [pltpu.VMEM((epw,), jnp.float32) for _ in range(7)]
        + [pltpu.VMEM((epw,), jnp.float32),
           pltpu.VMEM((4, epw), jnp.float32),
           pltpu.SemaphoreType.DMA],
    )
    def k(px_hbm, py_hbm, pz_hbm, pw_hbm, dst_hbm, pe_hbm, rel_hbm,
          idx_v, pjx_v, pjy_v, pjz_v, pjw_v, pix_v, piy_v, piz_v,
          pe_v, rel_v, sem):
        base = _worker_base(epw)
        pltpu.sync_copy(dst_hbm.at[pl.ds(base, epw)], idx_v)
        pltpu.sync_copy(px_hbm.at[pl.ds(base, epw)], pix_v)
        pltpu.sync_copy(py_hbm.at[pl.ds(base, epw)], piy_v)
        pltpu.sync_copy(pz_hbm.at[pl.ds(base, epw)], piz_v)
        cps = []
        for kk in range(nch):
            isl = idx_v.at[pl.ds(kk * CH, CH)]
            osl = pl.ds(kk * CH, CH)
            for tab, dv in ((px_hbm, pjx_v), (py_hbm, pjy_v),
                            (pz_hbm, pjz_v), (pw_hbm, pjw_v)):
                cps.append(pltpu.async_copy(tab.at[isl], dv.at[osl], sem))
        for cp in cps:
            cp.wait()

        def chunk(j, carry):
            r = j * 16
            sl = pl.ds(r, 16)
            rows = lax.iota(jnp.int32, 16) + r
            rx = pjx_v[sl] - pix_v[sl]
            ry = pjy_v[sl] - piy_v[sl]
            rz = pjz_v[sl] - piz_v[sl]
            pe = jnp.maximum(jnp.maximum(jnp.abs(rx), jnp.abs(ry)), jnp.abs(rz))
            pe = jnp.where(rows + base < e_real, pe, 0.0)
            pe_v[sl] = pe
            rel_v[0, sl] = rx
            rel_v[1, sl] = ry
            rel_v[2, sl] = rz
            rel_v[3, sl] = pjw_v[sl]
            return carry

        lax.fori_loop(0, epw // 16, chunk, 0)
        pltpu.sync_copy(pe_v, pe_hbm.at[pl.ds(base, epw)])
        pltpu.sync_copy(rel_v, rel_hbm.at[:, pl.ds(base, epw)])

    return k(*pos_cols, dst_p)


def _segment_max(src_p, pe, ep):
    npw = ep // NW
    qn = 4
    qs = ep // qn

    @functools.partial(
        pl.kernel,
        mesh=_sc_mesh(),
        compiler_params=pltpu.CompilerParams(needs_layout_passes=False),
        out_type=jax.ShapeDtypeStruct((ep,), jnp.float32),
        scratch_types=[pltpu.VMEM((qs,), jnp.int32),
                       pltpu.VMEM((qs,), jnp.int32),
                       pltpu.VMEM((qs,), jnp.float32),
                       pltpu.VMEM((qs,), jnp.float32),
                       pltpu.VMEM((32 * npw,), jnp.float32),
                       pltpu.VMEM((npw,), jnp.float32),
                       pltpu.SemaphoreType.DMA],
    )
    def k(src_hbm, pe_hbm, seg_hbm, src0_v, src1_v, pe0_v, pe1_v,
          seg32_v, seg_v, sem):
        n0 = _worker_base(npw)
        iota = lax.iota(jnp.int32, 16)
        plane_a = iota * npw
        plane_b = plane_a + 16 * npw
        srcs = (src0_v, src1_v)
        pes = (pe0_v, pe1_v)

        def zi(j, carry):
            seg32_v[pl.ds(j * 16, 16)] = jnp.zeros((16,), jnp.float32)
            return carry

        cps = {0: (pltpu.async_copy(src_hbm.at[pl.ds(0, qs)], src0_v, sem),
                   pltpu.async_copy(pe_hbm.at[pl.ds(0, qs)], pe0_v, sem))}
        lax.fori_loop(0, 2 * npw, zi, 0)

        def scat(sv, pv):
            def chunk(j, carry):
                sla = pl.ds(j * 32, 16)
                slb = pl.ds(j * 32 + 16, 16)
                la = sv[sla] - n0
                lb = sv[slb] - n0
                va = pv[sla]
                vb = pv[slb]
                ma = (la >= 0) & (la < npw)
                mb = (lb >= 0) & (lb < npw)
                ia = jnp.clip(la, 0, npw - 1) + plane_a
                ib = jnp.clip(lb, 0, npw - 1) + plane_b
                oa = plsc.load_gather(seg32_v, [ia])
                ob = plsc.load_gather(seg32_v, [ib])
                plsc.store_scatter(seg32_v, [ia], jnp.maximum(oa, va), mask=ma)
                plsc.store_scatter(seg32_v, [ib], jnp.maximum(ob, vb), mask=mb)
                return carry

            lax.fori_loop(0, qs // 32, chunk, 0)

        for q in range(qn):
            if q + 1 < qn:
                b = (q + 1) % 2
                cps[q + 1] = (
                    pltpu.async_copy(src_hbm.at[pl.ds((q + 1) * qs, qs)],
                                     srcs[b], sem),
                    pltpu.async_copy(pe_hbm.at[pl.ds((q + 1) * qs, qs)],
                                     pes[b], sem))
            cps[q][0].wait()
            cps[q][1].wait()
            scat(srcs[q % 2], pes[q % 2])

        def red(j, carry):
            acc = seg32_v[pl.ds(j * 16, 16)]
            for p in range(1, 32):
                acc = jnp.maximum(acc, seg32_v[pl.ds(p * npw + j * 16, 16)])
            seg_v[pl.ds(j * 16, 16)] = acc
            return carry

        lax.fori_loop(0, npw // 16, red, 0)
        pltpu.sync_copy(seg_v, seg_hbm.at[pl.ds(n0, npw)])

    return k(src_p, pe)


def _edge_assemble(seg, src_p, dst_p, rel, v_nodes, ep):
    epw = ep // NW
    nch = epw // CH
    nb = 7
    h = v_nodes.shape[1]

    @functools.partial(
        pl.kernel,
        mesh=_sc_mesh(),
        compiler_params=pltpu.CompilerParams(needs_layout_passes=False),
        out_type=[jax.ShapeDtypeStruct((4, ep), jnp.float32),
                  jax.ShapeDtypeStruct((ep, h), jnp.float32)],
        scratch_types=[pltpu.VMEM((epw,), jnp.int32),
                       pltpu.VMEM((epw,), jnp.int32),
                       pltpu.VMEM((epw,), jnp.float32),
                       pltpu.VMEM((4, epw), jnp.float32),
                       pltpu.VMEM((4, epw), jnp.float32)]
        + [pltpu.VMEM((CH, h), jnp.float32) for _ in range(nb)]
        + [pltpu.SemaphoreType.DMA,
           pltpu.SemaphoreType.DMA,
           pltpu.SemaphoreType.DMA],
    )
    def k(seg_hbm, src_hbm, dst_hbm, rel_hbm, v_hbm, geom_hbm, vg_hbm,
          sidx_v, didx_v, maxd_v, rel_v, geom_v, *bufs_and_sems):
        bufs = bufs_and_sems[:nb]
        gsem, osem, msem = bufs_and_sems[nb:]
        base = _worker_base(epw)
        pltpu.sync_copy(dst_hbm.at[pl.ds(base, epw)], didx_v)

        gcp = [pltpu.async_copy(v_hbm.at[didx_v.at[pl.ds(kk * CH, CH)]],
                                bufs[kk % nb], gsem)
               for kk in range(nb)]
        gcp += [None] * (nch - nb)

        pltpu.sync_copy(src_hbm.at[pl.ds(base, epw)], sidx_v)
        mcp = [pltpu.async_copy(seg_hbm.at[sidx_v.at[pl.ds(kk * CH, CH)]],
                                maxd_v.at[pl.ds(kk * CH, CH)], msem)
               for kk in range(nch)]
        pltpu.sync_copy(rel_hbm.at[:, pl.ds(base, epw)], rel_v)
        for cp in mcp:
            cp.wait()

        def chunk(j, carry):
            sl = pl.ds(j * 16, 16)
            inv = 1.0 / jnp.maximum(maxd_v[sl], 1e-8)
            geom_v[0, sl] = rel_v[0, sl] * inv
            geom_v[1, sl] = rel_v[1, sl] * inv
            geom_v[2, sl] = rel_v[2, sl] * inv
            geom_v[3, sl] = rel_v[3, sl]
            return carry

        lax.fori_loop(0, epw // 16, chunk, 0)
        pltpu.sync_copy(geom_v, geom_hbm.at[:, pl.ds(base, epw)])

        ocp = {}
        for kk in range(nch):
            gcp[kk].wait()
            ocp[kk] = pltpu.async_copy(
                bufs[kk % nb], vg_hbm.at[pl.ds(base + kk * CH, CH)], osem)
            prev = kk - 1
            if prev >= 0 and prev + nb < nch:
                ocp[prev].wait()
                gcp[prev + nb] = pltpu.async_copy(
                    v_hbm.at[didx_v.at[pl.ds((prev + nb) * CH, CH)]],
                    bufs[prev % nb], gsem)
        for kk in range(max(nch - nb, 0), nch):
            if kk >= 0:
                ocp[kk].wait()

    return k(seg, src_p, dst_p, rel, v_nodes)


def _h_block(vg_ref, g_ref, w1g_ref):
    acc = vg_ref[...]
    acc += lax.dot_general(g_ref[...], w1g_ref[...],
                           (((0,), (0,)), ((), ())),
                           preferred_element_type=jnp.float32)
    return jnp.maximum(acc, 0.0)


def _stats(vg, geom, W1g, e_real):
    ep, hp = vg.shape
    h = hp
    nb_grid = pl.cdiv(e_real, BLKS)

    def body(vg_ref, g_ref, w1g_ref, s_ref, q_ref):
        i = pl.program_id(0)
        acc = _h_block(vg_ref, g_ref, w1g_ref)
        rows = i * BLKS + lax.broadcasted_iota(jnp.int32, (BLKS, 1), 0)
        acc = jnp.where(rows < e_real, acc, 0.0)

        @pl.when(i == 0)
        def _():
            s_ref[...] = jnp.zeros_like(s_ref)
            q_ref[...] = jnp.zeros_like(q_ref)

        s_ref[...] += jnp.sum(acc, axis=0, keepdims=True)
        q_ref[...] += jnp.sum(acc * acc, axis=0, keepdims=True)

    return pl.pallas_call(
        body,
        grid=(nb_grid,),
        in_specs=[
            pl.BlockSpec((BLKS, hp), lambda i: (i, 0)),
            pl.BlockSpec((4, BLKS), lambda i: (0, i)),
            pl.BlockSpec((4, h), lambda i: (0, 0)),
        ],
        out_specs=[
            pl.BlockSpec((1, h), lambda i: (0, 0)),
            pl.BlockSpec((1, h), lambda i: (0, 0)),
        ],
        out_shape=[jax.ShapeDtypeStruct((1, h), jnp.float32),
                   jax.ShapeDtypeStruct((1, h), jnp.float32)],
    )(vg, geom, W1g)


def _normalize(vg, geom, W1g, s, q, gamma, beta, e_real):
    ep, hp = vg.shape
    h = hp
    nb_grid = pl.cdiv(e_real, BLKS)
    inv_n = 1.0 / e_real

    def body(vg_ref, g_ref, w1g_ref, s_ref, q_ref, gam_ref, bet_ref, o_ref):
        acc = _h_block(vg_ref, g_ref, w1g_ref)
        mean = s_ref[...] * inv_n
        var = q_ref[...] * inv_n - mean * mean
        scale = gam_ref[...] / jnp.sqrt(var + 1e-5)
        o_ref[...] = (acc - mean) * scale + bet_ref[...]

    return pl.pallas_call(
        body,
        grid=(nb_grid,),
        in_specs=[
            pl.BlockSpec((BLKS, hp), lambda i: (i, 0)),
            pl.BlockSpec((4, BLKS), lambda i: (0, i)),
            pl.BlockSpec((4, h), lambda i: (0, 0)),
            pl.BlockSpec((1, h), lambda i: (0, 0)),
            pl.BlockSpec((1, h), lambda i: (0, 0)),
            pl.BlockSpec((1, h), lambda i: (0, 0)),
            pl.BlockSpec((1, h), lambda i: (0, 0)),
        ],
        out_specs=pl.BlockSpec((BLKS, h), lambda i: (i, 0)),
        out_shape=jax.ShapeDtypeStruct((e_real, h), jnp.float32),
    )(vg, geom, W1g, s, q, gamma, beta)


def kernel(x, pos, edge_index, Wq, bq, Wk, bk, Wv, bv, W1, b1, gamma, beta):
    n, c = x.shape
    e = edge_index.shape[1]
    h = W1.shape[1]

    src = edge_index[0].astype(jnp.int32)
    dst = edge_index[1].astype(jnp.int32)

    epw = -(-e // (NW * CH)) * CH
    ep = NW * epw
    src_p = jnp.concatenate([src, jnp.zeros((ep - e,), jnp.int32)])
    dst_p = jnp.concatenate([dst, jnp.zeros((ep - e,), jnp.int32)])
    zpad = jnp.zeros((ep - n,), jnp.float32)
    pos_cols = [jnp.concatenate([pos[:, i], zpad]) for i in range(4)]

    W1v = W1[:c]
    W1g = W1[c:]

    v_nodes = _node_values(x, Wv, bv[None, :], W1v, b1[None, :])
    pe, rel = _edge_geometry(pos_cols, dst_p, ep, e)
    seg = _segment_max(src_p, pe, ep)
    geom, vg = _edge_assemble(seg, src_p, dst_p, rel, v_nodes, ep)
    s, q = _stats(vg, geom, W1g, e)
    return _normalize(vg, geom, W1g, s, q, gamma[None, :], beta[None, :], e)
